# baseline (speedup 1.0000x reference)
import jax
import jax.numpy as jnp
from jax.experimental import pallas as pl
from jax.experimental.pallas import tpu as pltpu

N = 2048
CIN = 128
COUT = 128
NHALF = N // 2

_BLK = 512
_NB = N // _BLK
_MW = 128

_F32 = jnp.float32


def _prologue_kernel(x_ref, xr_ref, xrt_ref, wi_ref, ws_ref, wti_ref, wts_ref,
                     ai_ref, as_ref, ait_ref, ast_ref,
                     fci_ref, fcs_ref, gri_ref, grs_ref,
                     mi_ref, ms_ref, mli_ref, mls_ref,
                     bd_scr, bdt_scr):
    ones_col = jnp.ones((CIN, 1), _F32)

    def dot6(a, b):
        return jnp.dot(a, b, preferred_element_type=_F32,
                       precision=jax.lax.Precision.HIGHEST)

    def dot1(a, b):
        return jnp.dot(a, b, preferred_element_type=_F32)

    mi_ref[...] = jnp.zeros((N, _MW), _F32)
    ms_ref[...] = jnp.zeros((N, _MW), _F32)
    mli_ref[...] = jnp.zeros((N, _MW), _F32)
    mls_ref[...] = jnp.zeros((N, _MW), _F32)
    for (w_ref, wt_ref, a_ref, at_ref, fc_ref, gr_ref, m_ref, ml_ref) in (
        (wi_ref, wti_ref, ai_ref, ait_ref, fci_ref, gri_ref, mi_ref, mli_ref),
        (ws_ref, wts_ref, as_ref, ast_ref, fcs_ref, grs_ref, ms_ref, mls_ref),
    ):
        for j in range(2):
            bd_scr[...] = jnp.zeros((2 * CIN, 2 * CIN), _F32)
            bd_scr[0:CIN, 0:CIN] = w_ref[j]
            bd_scr[CIN:2 * CIN, CIN:2 * CIN] = w_ref[j]
            hr = dot1(xr_ref[...], bd_scr[...])
            fc_ref[pl.ds(j * NHALF, NHALF), :] = dot1(hr, a_ref[0:2 * CIN, :])
            bdt_scr[...] = jnp.zeros((2 * CIN, 2 * CIN), _F32)
            bdt_scr[0:CIN, 0:CIN] = wt_ref[j]
            bdt_scr[CIN:2 * CIN, CIN:2 * CIN] = wt_ref[j]
            hrt = dot1(bdt_scr[...], xrt_ref[...])
            gr_ref[:, pl.ds(j * NHALF, NHALF)] = dot1(
                at_ref[:, 2 * CIN:4 * CIN], hrt)
            h = dot1(x_ref[...], w_ref[j])
            hb = h.astype(jnp.bfloat16).astype(_F32)
            tcol = dot6(hb, ones_col)
            thi = tcol.astype(jnp.bfloat16).astype(_F32)
            m_ref[:, pl.ds(j, 1)] = thi
            ml_ref[:, pl.ds(j, 1)] = tcol - thi


def _stats_kernel(ld_ref, lu_ref, fci_ref, fcs_ref, gri_ref, grs_ref,
                  mi_ref, ms_ref, mli_ref, mls_ref, z_ref,
                  c1i_scr, c1s_scr, ri_scr, rs_scr, z0_scr):
    i = pl.program_id(0)

    def fam(mat, f_col, g_row, m_cols, m_lo):
        mask = mat != 0.0
        e = f_col + g_row
        e = jnp.maximum(e, 0.01 * e)
        ex = jnp.where(mask, jnp.exp(e), 0.0)
        s = jnp.sum(ex, axis=1, keepdims=True)
        recip = jnp.where(s > 0.0, 1.0 / jnp.maximum(s, 1e-30), 0.0)
        ab = (ex * recip).astype(jnp.bfloat16).astype(_F32)
        ad = (jnp.dot(ab, m_cols, preferred_element_type=_F32)
              + jnp.dot(ab, m_lo, preferred_element_type=_F32))
        z0p = jnp.sum(ad[:, 0:1], axis=0, keepdims=True)
        rblk = ad[:, 1:2]
        c1p = jax.lax.dot_general(jnp.ones((_BLK, 1), _F32), ab,
                                  (((0,), (0,)), ((), ())),
                                  preferred_element_type=_F32)
        return z0p, c1p, rblk

    z0i, c1pi, rbi = fam(ld_ref[...], fci_ref[...], gri_ref[...],
                         mi_ref[...], mli_ref[...])
    z0s, c1ps, rbs = fam(lu_ref[...], fcs_ref[...], grs_ref[...],
                         ms_ref[...], mls_ref[...])

    ri_scr[pl.ds(i * _BLK, _BLK), :] = rbi
    rs_scr[pl.ds(i * _BLK, _BLK), :] = rbs

    @pl.when(i == 0)
    def _():
        c1i_scr[...] = c1pi
        c1s_scr[...] = c1ps
        z0_scr[...] = z0i + z0s

    @pl.when(i > 0)
    def _():
        c1i_scr[...] += c1pi
        c1s_scr[...] += c1ps
        z0_scr[...] += z0i + z0s

    @pl.when(i == _NB - 1)
    def _():
        z1i = jnp.dot(c1i_scr[...], ri_scr[...], preferred_element_type=_F32,
                      precision=jax.lax.Precision.HIGHEST)
        z1s = jnp.dot(c1s_scr[...], rs_scr[...], preferred_element_type=_F32,
                      precision=jax.lax.Precision.HIGHEST)
        z_ref[...] = z0_scr[...] + z1i + z1s


def _mm_kernel(p_ref, x_ref, wh_ref, z_ref, out_ref, xw_scr):
    i = pl.program_id(0)

    @pl.when(i == 0)
    def _():
        xw_scr[...] = jnp.dot(x_ref[...], wh_ref[...], preferred_element_type=_F32)

    out_ref[...] = (jnp.dot(p_ref[...], xw_scr[...], preferred_element_type=_F32)
                    + z_ref[...])


def kernel(x, Lup, Ldown, P, weight_irr, weight_sol, weight_har, att_irr, att_sol):
    xr = x.reshape(NHALF, 2 * CIN)
    xrt = xr.T
    wti = jnp.transpose(weight_irr, (0, 2, 1))
    wts = jnp.transpose(weight_sol, (0, 2, 1))
    ait = att_irr.T
    ast = att_sol.T

    vec_shapes = (
        jax.ShapeDtypeStruct((N, 1), _F32),
        jax.ShapeDtypeStruct((N, 1), _F32),
        jax.ShapeDtypeStruct((1, N), _F32),
        jax.ShapeDtypeStruct((1, N), _F32),
        jax.ShapeDtypeStruct((N, _MW), _F32),
        jax.ShapeDtypeStruct((N, _MW), _F32),
        jax.ShapeDtypeStruct((N, _MW), _F32),
        jax.ShapeDtypeStruct((N, _MW), _F32),
    )
    fci, fcs, gri, grs, mi, ms, mli, mls = pl.pallas_call(
        _prologue_kernel,
        out_shape=vec_shapes,
        scratch_shapes=[
            pltpu.VMEM((2 * CIN, 2 * CIN), _F32),
            pltpu.VMEM((2 * CIN, 2 * CIN), _F32),
        ],
    )(x, xr, xrt, weight_irr, weight_sol, wti, wts,
      att_irr, att_sol, ait, ast)

    row_blk = pl.BlockSpec((_BLK, N), lambda i: (i, 0))
    col_blk = pl.BlockSpec((_BLK, 1), lambda i: (i, 0))
    full_row = pl.BlockSpec((1, N), lambda i: (0, 0))
    full_m = pl.BlockSpec((N, _MW), lambda i: (0, 0))

    z = pl.pallas_call(
        _stats_kernel,
        grid=(_NB,),
        in_specs=[row_blk, row_blk, col_blk, col_blk,
                  full_row, full_row, full_m, full_m, full_m, full_m],
        out_specs=pl.BlockSpec((1, 1), lambda i: (0, 0)),
        out_shape=jax.ShapeDtypeStruct((1, 1), _F32),
        scratch_shapes=[
            pltpu.VMEM((1, N), _F32),
            pltpu.VMEM((1, N), _F32),
            pltpu.VMEM((N, 1), _F32),
            pltpu.VMEM((N, 1), _F32),
            pltpu.VMEM((1, 1), _F32),
        ],
    )(Ldown, Lup, fci, fcs, gri, grs, mi, ms, mli, mls)

    out = pl.pallas_call(
        _mm_kernel,
        grid=(_NB,),
        in_specs=[row_blk,
                  pl.BlockSpec((N, CIN), lambda i: (0, 0)),
                  pl.BlockSpec((CIN, COUT), lambda i: (0, 0)),
                  pl.BlockSpec((1, 1), lambda i: (0, 0))],
        out_specs=pl.BlockSpec((_BLK, COUT), lambda i: (i, 0)),
        out_shape=jax.ShapeDtypeStruct((N, COUT), _F32),
        scratch_shapes=[pltpu.VMEM((N, COUT), _F32)],
    )(P, x, weight_har, z)

    return out


# speedup vs baseline: 1.1077x; 1.1077x over previous
import jax
import jax.numpy as jnp
from jax.experimental import pallas as pl
from jax.experimental.pallas import tpu as pltpu

N = 2048
CIN = 128
COUT = 128
NHALF = N // 2

_BLK = 512
_NB = N // _BLK
_MW = 8

_F32 = jnp.float32


def _prologue_kernel(x_ref, xr_ref, xrt_ref, wi_ref, ws_ref, wti_ref, wts_ref,
                     ai_ref, as_ref, ait_ref, ast_ref,
                     fci_ref, fcs_ref, gri_ref, grs_ref,
                     mi_ref, ms_ref, mli_ref, mls_ref,
                     bd_scr, bdt_scr):
    ones_col = jnp.ones((CIN, 1), _F32)

    def dot6(a, b):
        return jnp.dot(a, b, preferred_element_type=_F32,
                       precision=jax.lax.Precision.HIGHEST)

    def dot1(a, b):
        return jnp.dot(a, b, preferred_element_type=_F32)

    mi_ref[...] = jnp.zeros((N, _MW), jnp.bfloat16)
    ms_ref[...] = jnp.zeros((N, _MW), jnp.bfloat16)
    mli_ref[...] = jnp.zeros((N, _MW), jnp.bfloat16)
    mls_ref[...] = jnp.zeros((N, _MW), jnp.bfloat16)
    bd_scr[...] = jnp.zeros((2 * CIN, 2 * CIN), _F32)
    bdt_scr[...] = jnp.zeros((2 * CIN, 2 * CIN), _F32)
    for (w_ref, wt_ref, a_ref, at_ref, fc_ref, gr_ref, m_ref, ml_ref) in (
        (wi_ref, wti_ref, ai_ref, ait_ref, fci_ref, gri_ref, mi_ref, mli_ref),
        (ws_ref, wts_ref, as_ref, ast_ref, fcs_ref, grs_ref, ms_ref, mls_ref),
    ):
        for j in range(2):
            bd_scr[0:CIN, 0:CIN] = w_ref[j]
            bd_scr[CIN:2 * CIN, CIN:2 * CIN] = w_ref[j]
            hr = dot1(xr_ref[...], bd_scr[...])
            fc_ref[pl.ds(j * NHALF, NHALF), :] = dot1(hr, a_ref[0:2 * CIN, :])
            bdt_scr[0:CIN, 0:CIN] = wt_ref[j]
            bdt_scr[CIN:2 * CIN, CIN:2 * CIN] = wt_ref[j]
            hrt = dot1(bdt_scr[...], xrt_ref[...])
            gr_ref[:, pl.ds(j * NHALF, NHALF)] = dot1(
                at_ref[:, 2 * CIN:4 * CIN], hrt)
            h = dot1(x_ref[...], w_ref[j])
            hb = h.astype(jnp.bfloat16).astype(_F32)
            tcol = dot1(hb, ones_col)
            thi = tcol.astype(jnp.bfloat16)
            m_ref[:, pl.ds(j, 1)] = thi
            ml_ref[:, pl.ds(j, 1)] = (tcol - thi.astype(_F32)).astype(jnp.bfloat16)


def _stats_kernel(ld_ref, lu_ref, fci_ref, fcs_ref, gri_ref, grs_ref,
                  mi_ref, ms_ref, mli_ref, mls_ref, z_ref,
                  c1i_scr, c1s_scr, ri_scr, rs_scr, z0_scr):
    i = pl.program_id(0)

    def fam(mat, f_col, g_row, m_cols, m_lo):
        mask = mat != 0.0
        e = f_col + g_row
        e = jnp.maximum(e, 0.01 * e)
        ex = jnp.where(mask, jnp.exp(e), 0.0)
        s = jnp.sum(ex, axis=1, keepdims=True)
        recip = jnp.where(s > 0.0, 1.0 / jnp.maximum(s, 1e-30), 0.0)
        ab = (ex * recip).astype(jnp.bfloat16)
        ad = (jnp.dot(ab, m_cols, preferred_element_type=_F32)
              + jnp.dot(ab, m_lo, preferred_element_type=_F32))
        z0p = jnp.sum(ad[:, 0:1], axis=0, keepdims=True)
        rblk = ad[:, 1:2]
        c1p = jax.lax.dot_general(jnp.ones((_BLK, 1), jnp.bfloat16), ab,
                                  (((0,), (0,)), ((), ())),
                                  preferred_element_type=_F32)
        return z0p, c1p, rblk

    z0i, c1pi, rbi = fam(ld_ref[...], fci_ref[...], gri_ref[...],
                         mi_ref[...], mli_ref[...])
    z0s, c1ps, rbs = fam(lu_ref[...], fcs_ref[...], grs_ref[...],
                         ms_ref[...], mls_ref[...])

    ri_scr[pl.ds(i * _BLK, _BLK), :] = rbi
    rs_scr[pl.ds(i * _BLK, _BLK), :] = rbs

    @pl.when(i == 0)
    def _():
        c1i_scr[...] = c1pi
        c1s_scr[...] = c1ps
        z0_scr[...] = z0i + z0s

    @pl.when(i > 0)
    def _():
        c1i_scr[...] += c1pi
        c1s_scr[...] += c1ps
        z0_scr[...] += z0i + z0s

    @pl.when(i == _NB - 1)
    def _():
        z1i = jnp.dot(c1i_scr[...], ri_scr[...], preferred_element_type=_F32,
                      precision=jax.lax.Precision.HIGHEST)
        z1s = jnp.dot(c1s_scr[...], rs_scr[...], preferred_element_type=_F32,
                      precision=jax.lax.Precision.HIGHEST)
        z_ref[...] = z0_scr[...] + z1i + z1s


def _mm_kernel(p_ref, x_ref, wh_ref, z_ref, out_ref, xw_scr):
    i = pl.program_id(0)

    @pl.when(i == 0)
    def _():
        xw_scr[...] = jnp.dot(x_ref[...], wh_ref[...], preferred_element_type=_F32)

    out_ref[...] = (jnp.dot(p_ref[...], xw_scr[...], preferred_element_type=_F32)
                    + z_ref[...])


def kernel(x, Lup, Ldown, P, weight_irr, weight_sol, weight_har, att_irr, att_sol):
    xr = x.reshape(NHALF, 2 * CIN)
    xrt = xr.T
    wti = jnp.transpose(weight_irr, (0, 2, 1))
    wts = jnp.transpose(weight_sol, (0, 2, 1))
    ait = att_irr.T
    ast = att_sol.T

    vec_shapes = (
        jax.ShapeDtypeStruct((N, 1), _F32),
        jax.ShapeDtypeStruct((N, 1), _F32),
        jax.ShapeDtypeStruct((1, N), _F32),
        jax.ShapeDtypeStruct((1, N), _F32),
        jax.ShapeDtypeStruct((N, _MW), jnp.bfloat16),
        jax.ShapeDtypeStruct((N, _MW), jnp.bfloat16),
        jax.ShapeDtypeStruct((N, _MW), jnp.bfloat16),
        jax.ShapeDtypeStruct((N, _MW), jnp.bfloat16),
    )
    fci, fcs, gri, grs, mi, ms, mli, mls = pl.pallas_call(
        _prologue_kernel,
        out_shape=vec_shapes,
        scratch_shapes=[
            pltpu.VMEM((2 * CIN, 2 * CIN), _F32),
            pltpu.VMEM((2 * CIN, 2 * CIN), _F32),
        ],
    )(x, xr, xrt, weight_irr, weight_sol, wti, wts,
      att_irr, att_sol, ait, ast)

    row_blk = pl.BlockSpec((_BLK, N), lambda i: (i, 0))
    col_blk = pl.BlockSpec((_BLK, 1), lambda i: (i, 0))
    full_row = pl.BlockSpec((1, N), lambda i: (0, 0))
    full_m = pl.BlockSpec((N, _MW), lambda i: (0, 0))

    z = pl.pallas_call(
        _stats_kernel,
        grid=(_NB,),
        in_specs=[row_blk, row_blk, col_blk, col_blk,
                  full_row, full_row, full_m, full_m, full_m, full_m],
        out_specs=pl.BlockSpec((1, 1), lambda i: (0, 0)),
        out_shape=jax.ShapeDtypeStruct((1, 1), _F32),
        scratch_shapes=[
            pltpu.VMEM((1, N), _F32),
            pltpu.VMEM((1, N), _F32),
            pltpu.VMEM((N, 1), _F32),
            pltpu.VMEM((N, 1), _F32),
            pltpu.VMEM((1, 1), _F32),
        ],
    )(Ldown, Lup, fci, fcs, gri, grs, mi, ms, mli, mls)

    out = pl.pallas_call(
        _mm_kernel,
        grid=(_NB,),
        in_specs=[row_blk,
                  pl.BlockSpec((N, CIN), lambda i: (0, 0)),
                  pl.BlockSpec((CIN, COUT), lambda i: (0, 0)),
                  pl.BlockSpec((1, 1), lambda i: (0, 0))],
        out_specs=pl.BlockSpec((_BLK, COUT), lambda i: (i, 0)),
        out_shape=jax.ShapeDtypeStruct((N, COUT), _F32),
        scratch_shapes=[pltpu.VMEM((N, COUT), _F32)],
    )(P, x, weight_har, z)

    return out


# speedup vs baseline: 1.3226x; 1.1941x over previous
import jax
import jax.numpy as jnp
from jax.experimental import pallas as pl
from jax.experimental.pallas import tpu as pltpu

N = 2048
CIN = 128
COUT = 128
NHALF = N // 2

_BLK = 512
_NB = N // _BLK
_BLKM = 512
_NBM = N // _BLKM
_MW = 16

_F32 = jnp.float32
_BF16 = jnp.bfloat16


def _fused_kernel(ld_ref, lu_ref, p_ref, x_ref, xr_ref, xrt_ref,
                  wi_ref, ws_ref, wti_ref, wts_ref,
                  ai_ref, as_ref, ait_ref, ast_ref, wh_ref,
                  out_ref,
                  fci_scr, fcs_scr, gri_scr, grs_scr, mi_scr, ms_scr,
                  bd_scr, bdt_scr,
                  c1i_scr, c1s_scr, ri_scr, rs_scr, z_scr, xw_scr):
    i = pl.program_id(0)

    def dot1(a, b):
        return jnp.dot(a, b, preferred_element_type=_F32)

    @pl.when(i == 0)
    def _prologue():
        ones_col = jnp.ones((CIN, 1), _F32)
        bd_scr[...] = jnp.zeros((2 * CIN, 2 * CIN), _F32)
        bdt_scr[...] = jnp.zeros((2 * CIN, 2 * CIN), _F32)
        for (w_ref, wt_ref, a_ref, at_ref, fc_scr, gr_scr, m_scr) in (
            (wi_ref, wti_ref, ai_ref, ait_ref, fci_scr, gri_scr, mi_scr),
            (ws_ref, wts_ref, as_ref, ast_ref, fcs_scr, grs_scr, ms_scr),
        ):
            m_scr[...] = jnp.zeros((N, _MW), _BF16)
            for j in range(2):
                bd_scr[0:CIN, 0:CIN] = w_ref[j]
                bd_scr[CIN:2 * CIN, CIN:2 * CIN] = w_ref[j]
                hr = dot1(xr_ref[...], bd_scr[...])
                fc_scr[pl.ds(j * NHALF, NHALF), :] = dot1(
                    hr, a_ref[0:2 * CIN, :])
                bdt_scr[0:CIN, 0:CIN] = wt_ref[j]
                bdt_scr[CIN:2 * CIN, CIN:2 * CIN] = wt_ref[j]
                hrt = dot1(bdt_scr[...], xrt_ref[...])
                gr_scr[:, pl.ds(j * NHALF, NHALF)] = dot1(
                    at_ref[:, 2 * CIN:4 * CIN], hrt)
                h = dot1(x_ref[...], w_ref[j])
                hb = h.astype(_BF16).astype(_F32)
                tcol = dot1(hb, ones_col)
                thi = tcol.astype(_BF16)
                m_scr[:, pl.ds(j, 1)] = thi
                m_scr[:, pl.ds(2 + j, 1)] = (tcol
                                             - thi.astype(_F32)).astype(_BF16)

    @pl.when(i < _NB)
    def _stats():
        def fam(mat, f_col, g_row, m_cols):
            mask = mat != 0.0
            e = f_col + g_row
            e = jnp.maximum(e, 0.01 * e)
            ex = jnp.where(mask, jnp.exp(e), 0.0)
            s = jnp.sum(ex, axis=1, keepdims=True)
            recip = jnp.where(s > 0.0, 1.0 / jnp.maximum(s, 1e-30), 0.0)
            ab = (ex * recip).astype(_BF16)
            ad = jnp.dot(ab, m_cols, preferred_element_type=_F32)
            z0p = jnp.sum(ad[:, 0:1] + ad[:, 2:3], axis=0, keepdims=True)
            rblk = ad[:, 1:2] + ad[:, 3:4]
            c1p = jax.lax.dot_general(jnp.ones((_BLK, 1), _BF16), ab,
                                      (((0,), (0,)), ((), ())),
                                      preferred_element_type=_F32)
            return z0p, c1p, rblk

        z0i, c1pi, rbi = fam(ld_ref[...], fci_scr[pl.ds(i * _BLK, _BLK), :],
                             gri_scr[...], mi_scr[...])
        z0s, c1ps, rbs = fam(lu_ref[...], fcs_scr[pl.ds(i * _BLK, _BLK), :],
                             grs_scr[...], ms_scr[...])

        ri_scr[pl.ds(i * _BLK, _BLK), :] = rbi
        rs_scr[pl.ds(i * _BLK, _BLK), :] = rbs

        @pl.when(i == 0)
        def _():
            c1i_scr[...] = c1pi
            c1s_scr[...] = c1ps
            z_scr[...] = z0i + z0s

        @pl.when(i > 0)
        def _():
            c1i_scr[...] += c1pi
            c1s_scr[...] += c1ps
            z_scr[...] += z0i + z0s

        @pl.when(i == _NB - 1)
        def _():
            z1i = jnp.dot(c1i_scr[...], ri_scr[...],
                          preferred_element_type=_F32,
                          precision=jax.lax.Precision.HIGHEST)
            z1s = jnp.dot(c1s_scr[...], rs_scr[...],
                          preferred_element_type=_F32,
                          precision=jax.lax.Precision.HIGHEST)
            z_scr[...] += z1i + z1s
            xw_scr[...] = jnp.dot(x_ref[...], wh_ref[...],
                                  preferred_element_type=_F32)

    @pl.when(i >= _NB)
    def _mm():
        out_ref[...] = (jnp.dot(p_ref[...], xw_scr[...],
                                preferred_element_type=_F32) + z_scr[...])


def kernel(x, Lup, Ldown, P, weight_irr, weight_sol, weight_har, att_irr, att_sol):
    xr = x.reshape(NHALF, 2 * CIN)
    xrt = xr.T
    wti = jnp.transpose(weight_irr, (0, 2, 1))
    wts = jnp.transpose(weight_sol, (0, 2, 1))
    ait = att_irr.T
    ast = att_sol.T

    def cst(shape):
        return pl.BlockSpec(shape, lambda i: tuple(0 for _ in shape))

    out = pl.pallas_call(
        _fused_kernel,
        grid=(_NB + _NBM,),
        in_specs=[
            pl.BlockSpec((_BLK, N), lambda i: (jnp.minimum(i, _NB - 1), 0)),
            pl.BlockSpec((_BLK, N), lambda i: (jnp.minimum(i, _NB - 1), 0)),
            pl.BlockSpec((_BLKM, N),
                         lambda i: (jnp.maximum(i - _NB, 0), 0)),
            cst((N, CIN)),
            cst((NHALF, 2 * CIN)),
            cst((2 * CIN, NHALF)),
            cst((2, CIN, COUT)),
            cst((2, CIN, COUT)),
            cst((2, COUT, CIN)),
            cst((2, COUT, CIN)),
            cst((4 * CIN, 1)),
            cst((4 * CIN, 1)),
            cst((1, 4 * CIN)),
            cst((1, 4 * CIN)),
            cst((CIN, COUT)),
        ],
        out_specs=pl.BlockSpec((_BLKM, COUT),
                               lambda i: (jnp.maximum(i - _NB, 0), 0)),
        out_shape=jax.ShapeDtypeStruct((N, COUT), _F32),
        scratch_shapes=[
            pltpu.VMEM((N, 1), _F32),
            pltpu.VMEM((N, 1), _F32),
            pltpu.VMEM((1, N), _F32),
            pltpu.VMEM((1, N), _F32),
            pltpu.VMEM((N, _MW), _BF16),
            pltpu.VMEM((N, _MW), _BF16),
            pltpu.VMEM((2 * CIN, 2 * CIN), _F32),
            pltpu.VMEM((2 * CIN, 2 * CIN), _F32),
            pltpu.VMEM((1, N), _F32),
            pltpu.VMEM((1, N), _F32),
            pltpu.VMEM((N, 1), _F32),
            pltpu.VMEM((N, 1), _F32),
            pltpu.VMEM((1, 1), _F32),
            pltpu.VMEM((N, COUT), _F32),
        ],
    )(Ldown, Lup, P, x, xr, xrt, weight_irr, weight_sol, wti, wts,
      att_irr, att_sol, ait, ast, weight_har)

    return out


# speedup vs baseline: 1.3361x; 1.0102x over previous
import jax
import jax.numpy as jnp
from jax.experimental import pallas as pl
from jax.experimental.pallas import tpu as pltpu

N = 2048
CIN = 128
COUT = 128
NHALF = N // 2

_BLK = 256
_NB = N // _BLK
_BLKM = 512
_NBM = N // _BLKM
_MW = 16

_F32 = jnp.float32
_BF16 = jnp.bfloat16


def _fused_kernel(ld_ref, lu_ref, p_ref, x_ref, xr_ref, xrt_ref,
                  wi_ref, ws_ref, wti_ref, wts_ref,
                  ai_ref, as_ref, ait_ref, ast_ref, wh_ref,
                  out_ref,
                  fci_scr, fcs_scr, gri_scr, grs_scr, mi_scr, ms_scr,
                  bd_scr, bdt_scr,
                  c1i_scr, c1s_scr, ri_scr, rs_scr, z_scr, xw_scr):
    i = pl.program_id(0)

    def dot1(a, b):
        return jnp.dot(a, b, preferred_element_type=_F32)

    @pl.when(i == 0)
    def _prologue():
        ones_col = jnp.ones((CIN, 1), _F32)
        bd_scr[...] = jnp.zeros((2 * CIN, 2 * CIN), _F32)
        bdt_scr[...] = jnp.zeros((2 * CIN, 2 * CIN), _F32)
        for (w_ref, wt_ref, a_ref, at_ref, fc_scr, gr_scr, m_scr) in (
            (wi_ref, wti_ref, ai_ref, ait_ref, fci_scr, gri_scr, mi_scr),
            (ws_ref, wts_ref, as_ref, ast_ref, fcs_scr, grs_scr, ms_scr),
        ):
            m_scr[...] = jnp.zeros((N, _MW), _BF16)
            for j in range(2):
                bd_scr[0:CIN, 0:CIN] = w_ref[j]
                bd_scr[CIN:2 * CIN, CIN:2 * CIN] = w_ref[j]
                hr = dot1(xr_ref[...], bd_scr[...])
                fc_scr[pl.ds(j * NHALF, NHALF), :] = dot1(
                    hr, a_ref[0:2 * CIN, :])
                bdt_scr[0:CIN, 0:CIN] = wt_ref[j]
                bdt_scr[CIN:2 * CIN, CIN:2 * CIN] = wt_ref[j]
                hrt = dot1(bdt_scr[...], xrt_ref[...])
                gr_scr[:, pl.ds(j * NHALF, NHALF)] = dot1(
                    at_ref[:, 2 * CIN:4 * CIN], hrt)
                h = dot1(x_ref[...], w_ref[j])
                hb = h.astype(_BF16).astype(_F32)
                tcol = dot1(hb, ones_col)
                thi = tcol.astype(_BF16)
                m_scr[:, pl.ds(j, 1)] = thi
                m_scr[:, pl.ds(2 + j, 1)] = (tcol
                                             - thi.astype(_F32)).astype(_BF16)

    @pl.when(i < _NB)
    def _stats():
        def fam(mat, f_col, g_row, m_cols):
            mask = mat != 0.0
            e = f_col + g_row
            e = jnp.maximum(e, 0.01 * e)
            ex = jnp.where(mask, jnp.exp(e), 0.0)
            s = jnp.sum(ex, axis=1, keepdims=True)
            recip = jnp.where(s > 0.0, 1.0 / jnp.maximum(s, 1e-30), 0.0)
            ab = (ex * recip).astype(_BF16)
            ad = jnp.dot(ab, m_cols, preferred_element_type=_F32)
            z0p = jnp.sum(ad[:, 0:1] + ad[:, 2:3], axis=0, keepdims=True)
            rblk = ad[:, 1:2] + ad[:, 3:4]
            c1p = jax.lax.dot_general(jnp.ones((_BLK, 1), _BF16), ab,
                                      (((0,), (0,)), ((), ())),
                                      preferred_element_type=_F32)
            return z0p, c1p, rblk

        z0i, c1pi, rbi = fam(ld_ref[...], fci_scr[pl.ds(i * _BLK, _BLK), :],
                             gri_scr[...], mi_scr[...])
        z0s, c1ps, rbs = fam(lu_ref[...], fcs_scr[pl.ds(i * _BLK, _BLK), :],
                             grs_scr[...], ms_scr[...])

        ri_scr[pl.ds(i * _BLK, _BLK), :] = rbi
        rs_scr[pl.ds(i * _BLK, _BLK), :] = rbs

        @pl.when(i == 0)
        def _():
            c1i_scr[...] = c1pi
            c1s_scr[...] = c1ps
            z_scr[...] = z0i + z0s

        @pl.when(i > 0)
        def _():
            c1i_scr[...] += c1pi
            c1s_scr[...] += c1ps
            z_scr[...] += z0i + z0s

        @pl.when(i == _NB - 1)
        def _():
            z1i = jnp.dot(c1i_scr[...], ri_scr[...],
                          preferred_element_type=_F32,
                          precision=jax.lax.Precision.HIGHEST)
            z1s = jnp.dot(c1s_scr[...], rs_scr[...],
                          preferred_element_type=_F32,
                          precision=jax.lax.Precision.HIGHEST)
            z_scr[...] += z1i + z1s
            xw_scr[...] = jnp.dot(x_ref[...], wh_ref[...],
                                  preferred_element_type=_F32)

    @pl.when(i >= _NB)
    def _mm():
        out_ref[...] = (jnp.dot(p_ref[...], xw_scr[...],
                                preferred_element_type=_F32) + z_scr[...])


def kernel(x, Lup, Ldown, P, weight_irr, weight_sol, weight_har, att_irr, att_sol):
    xr = x.reshape(NHALF, 2 * CIN)
    xrt = xr.T
    wti = jnp.transpose(weight_irr, (0, 2, 1))
    wts = jnp.transpose(weight_sol, (0, 2, 1))
    ait = att_irr.T
    ast = att_sol.T

    def cst(shape):
        return pl.BlockSpec(shape, lambda i: tuple(0 for _ in shape))

    out = pl.pallas_call(
        _fused_kernel,
        grid=(_NB + _NBM,),
        in_specs=[
            pl.BlockSpec((_BLK, N), lambda i: (jnp.minimum(i, _NB - 1), 0)),
            pl.BlockSpec((_BLK, N), lambda i: (jnp.minimum(i, _NB - 1), 0)),
            pl.BlockSpec((_BLKM, N),
                         lambda i: (jnp.maximum(i - _NB, 0), 0)),
            cst((N, CIN)),
            cst((NHALF, 2 * CIN)),
            cst((2 * CIN, NHALF)),
            cst((2, CIN, COUT)),
            cst((2, CIN, COUT)),
            cst((2, COUT, CIN)),
            cst((2, COUT, CIN)),
            cst((4 * CIN, 1)),
            cst((4 * CIN, 1)),
            cst((1, 4 * CIN)),
            cst((1, 4 * CIN)),
            cst((CIN, COUT)),
        ],
        out_specs=pl.BlockSpec((_BLKM, COUT),
                               lambda i: (jnp.maximum(i - _NB, 0), 0)),
        out_shape=jax.ShapeDtypeStruct((N, COUT), _F32),
        scratch_shapes=[
            pltpu.VMEM((N, 1), _F32),
            pltpu.VMEM((N, 1), _F32),
            pltpu.VMEM((1, N), _F32),
            pltpu.VMEM((1, N), _F32),
            pltpu.VMEM((N, _MW), _BF16),
            pltpu.VMEM((N, _MW), _BF16),
            pltpu.VMEM((2 * CIN, 2 * CIN), _F32),
            pltpu.VMEM((2 * CIN, 2 * CIN), _F32),
            pltpu.VMEM((1, N), _F32),
            pltpu.VMEM((1, N), _F32),
            pltpu.VMEM((N, 1), _F32),
            pltpu.VMEM((N, 1), _F32),
            pltpu.VMEM((1, 1), _F32),
            pltpu.VMEM((N, COUT), _F32),
        ],
    )(Ldown, Lup, P, x, xr, xrt, weight_irr, weight_sol, wti, wts,
      att_irr, att_sol, ait, ast, weight_har)

    return out
